# baseline (speedup 1.0000x reference)
import jax
import jax.numpy as jnp
from jax import lax
from jax.experimental import pallas as pl
from jax.experimental.pallas import tpu as pltpu
from jax.experimental.pallas import tpu_sc as plsc

D = 128
NS = 16
NC = 2
GR = 2
EG = GR * 128
PAD_ROWS = 2560


def _sc_mesh():
    return plsc.VectorSubcoreMesh(core_axis_name="c", subcore_axis_name="s")


def _sc_params():
    return pltpu.CompilerParams(use_tc_tiling_on_sc=False)


def _counts_call(tpn, hpn, zc):
    n = zc.shape[0] * NS
    nper = n // NS
    nbatch = PAD_ROWS // 8 // NS

    def body(t_ref, h_ref, zc_ref, ct_ref, ch_ref, acc, idxb, ones, sem):
        cid = lax.axis_index("c")
        sid = lax.axis_index("s")

        def fill(i, _):
            for jj in range(8):
                ones[i, pl.ds(jj * 16, 16)] = jnp.full((16,), 1.0, jnp.float32)
            return 0

        lax.fori_loop(0, 128, fill, 0)
        pltpu.sync_copy(zc_ref, acc.at[pl.ds(sid * nper, nper)])
        plsc.subcore_barrier()

        def stream(idx_ref):
            def batch(b, _):
                r0 = (b * NS + sid) * 8
                pltpu.sync_copy(idx_ref.at[pl.ds(r0, 8)], idxb)
                cps = [
                    pltpu.async_copy(ones, acc.at[idxb.at[j]], sem, add=True)
                    for j in range(8)
                ]
                for cp in cps:
                    cp.wait()
                return 0

            lax.fori_loop(0, nbatch, batch, 0)

        @pl.when(cid == 0)
        def _():
            stream(t_ref)

        @pl.when(cid == 1)
        def _():
            stream(h_ref)

        plsc.subcore_barrier()

        @pl.when(cid == 0)
        def _():
            pltpu.sync_copy(
                acc.at[pl.ds(sid * nper, nper)], ct_ref.at[pl.ds(sid * nper, nper)]
            )

        @pl.when(cid == 1)
        def _():
            pltpu.sync_copy(
                acc.at[pl.ds(sid * nper, nper)], ch_ref.at[pl.ds(sid * nper, nper)]
            )

    out = jax.ShapeDtypeStruct((n, 128), jnp.float32)
    return pl.kernel(
        body,
        out_type=(out, out),
        mesh=_sc_mesh(),
        compiler_params=_sc_params(),
        scratch_types=[
            pltpu.VMEM_SHARED((n + 16, 128), jnp.float32),
            pltpu.VMEM((8, 128), jnp.int32),
            pltpu.VMEM((128, 128), jnp.float32),
            pltpu.SemaphoreType.DMA,
        ],
    )(tpn, hpn, zc)


def _gather_call(h, hp0, tp0, r_rows, prow):
    n, d = h.shape
    m = r_rows * 128
    ng = (r_rows + GR - 1) // GR
    nloop = prow // GR // NS

    def body(h_ref, hidx_ref, tidx_ref, hh_ref, ht_ref,
             idx0, idx1, rows0, rows1, sem0, sem1):
        cid = lax.axis_index("c")
        sid = lax.axis_index("s")

        def stream(idx_ref, out_ref):
            g0_ = sid * nloop
            bufs = ((idx0, rows0, sem0), (idx1, rows1, sem1))

            def prep(i, pr):
                idxb, rows, sem = bufs[pr]
                g = g0_ + i
                pltpu.sync_copy(idx_ref.at[pl.ds(g * GR, GR)], idxb)
                for j in range(GR):
                    pltpu.async_copy(
                        h_ref.at[idxb.at[j]],
                        rows.at[pl.ds(j * 128, 128)],
                        sem,
                    )

            def finish(i, pr):
                idxb, rows, sem = bufs[pr]
                g = g0_ + i
                for j in range(GR):
                    pltpu.make_async_copy(
                        h_ref.at[idxb.at[j]],
                        rows.at[pl.ds(j * 128, 128)],
                        sem,
                    ).wait()

                @pl.when(g < ng)
                def _():
                    pltpu.sync_copy(rows, out_ref.at[pl.ds(g * EG, EG)])

            prep(0, 0)

            def it(k, _):
                i0 = k * 2
                prep(i0 + 1, 1)
                finish(i0, 0)

                @pl.when(i0 + 2 < nloop)
                def _():
                    prep(i0 + 2, 0)

                finish(i0 + 1, 1)
                return 0

            lax.fori_loop(0, nloop // 2, it, 0)

        @pl.when(cid == 0)
        def _():
            stream(hidx_ref, hh_ref)

        @pl.when(cid == 1)
        def _():
            stream(tidx_ref, ht_ref)

    out = jax.ShapeDtypeStruct((m, d), jnp.float32)
    return pl.kernel(
        body,
        out_type=(out, out),
        mesh=_sc_mesh(),
        compiler_params=_sc_params(),
        scratch_types=[
            pltpu.VMEM((GR, 128), jnp.int32),
            pltpu.VMEM((GR, 128), jnp.int32),
            pltpu.VMEM((EG, d), jnp.float32),
            pltpu.VMEM((EG, d), jnp.float32),
            pltpu.SemaphoreType.DMA,
            pltpu.SemaphoreType.DMA,
        ],
    )(h, hp0, tp0)


def _scatter_m_call(mf, mb, tpn, hpn, zrows, r_rows, prow):
    m, d = mf.shape
    n = zrows.shape[0] * NS
    nper = n // NS
    nloop = prow // NS
    t32 = tpn.reshape(prow * 4, 32)
    h32 = hpn.reshape(prow * 4, 32)

    def body(mf_ref, mb_ref, t_ref, h_ref, z_ref, af_ref, ab_ref,
             acc, idx0, idx1, rows0, rows1, lsem0, lsem1, ssem0, ssem1):
        cid = lax.axis_index("c")
        sid = lax.axis_index("s")
        pltpu.sync_copy(z_ref, acc.at[pl.ds(sid * nper, nper)])
        plsc.subcore_barrier()

        def stream(src_ref, idx_ref):
            g0_ = sid * nloop
            bufs = (
                (idx0, rows0, lsem0, ssem0),
                (idx1, rows1, lsem1, ssem1),
            )

            def prep(i, pr):
                ib, rows, lsem, _ = bufs[pr]
                g = g0_ + i
                r = jnp.where(g < r_rows, g, 0)
                pltpu.sync_copy(idx_ref.at[pl.ds(4 * g, 4)], ib)
                pltpu.async_copy(src_ref.at[pl.ds(r * 128, 128)], rows, lsem)

            def launch(i, pr):
                ib, rows, lsem, ssem = bufs[pr]
                pltpu.make_async_copy(
                    src_ref.at[pl.ds(0, 128)], rows, lsem
                ).wait()
                for j in range(4):
                    pltpu.async_copy(
                        rows.at[pl.ds(j * 32, 32)],
                        acc.at[ib.at[j]],
                        ssem,
                        add=True,
                    )

            def drain(pr):
                ib, rows, _, ssem = bufs[pr]
                for j in range(4):
                    pltpu.make_async_copy(
                        rows.at[pl.ds(j * 32, 32)], acc.at[ib.at[j]], ssem
                    ).wait()

            prep(0, 0)

            def it(k, _):
                i0 = k * 2
                launch(i0, 0)

                @pl.when(k > 0)
                def _():
                    drain(1)

                prep(i0 + 1, 1)
                drain(0)

                @pl.when(i0 + 2 < nloop)
                def _():
                    prep(i0 + 2, 0)

                launch(i0 + 1, 1)
                return 0

            lax.fori_loop(0, nloop // 2, it, 0)
            drain(1)

        @pl.when(cid == 0)
        def _():
            stream(mf_ref, t_ref)

        @pl.when(cid == 1)
        def _():
            stream(mb_ref, h_ref)

        plsc.subcore_barrier()

        @pl.when(cid == 0)
        def _():
            pltpu.sync_copy(
                acc.at[pl.ds(sid * nper, nper)], af_ref.at[pl.ds(sid * nper, nper)]
            )

        @pl.when(cid == 1)
        def _():
            pltpu.sync_copy(
                acc.at[pl.ds(sid * nper, nper)], ab_ref.at[pl.ds(sid * nper, nper)]
            )

    out = jax.ShapeDtypeStruct((n, d), jnp.float32)
    return pl.kernel(
        body,
        out_type=(out, out),
        mesh=_sc_mesh(),
        compiler_params=_sc_params(),
        scratch_types=[
            pltpu.VMEM_SHARED((n + 16, d), jnp.float32),
            pltpu.VMEM((4, 32), jnp.int32),
            pltpu.VMEM((4, 32), jnp.int32),
            pltpu.VMEM((128, d), jnp.float32),
            pltpu.VMEM((128, d), jnp.float32),
            pltpu.SemaphoreType.DMA,
            pltpu.SemaphoreType.DMA,
            pltpu.SemaphoreType.DMA,
            pltpu.SemaphoreType.DMA,
        ],
    )(mf, mb, t32, h32, zrows)


def _leaky(x):
    return jnp.where(x >= 0, x, 0.01 * x)


def _ln(x, g, b):
    mu = jnp.mean(x, axis=-1, keepdims=True)
    var = jnp.mean((x - mu) ** 2, axis=-1, keepdims=True)
    return (x - mu) * lax.rsqrt(var + 1e-5) * g + b


def _edge_msg_call(e, hh, ht, wpre, wf, wb, be, ge, bee, bf, bb):
    m, d = e.shape
    bm = 3200

    def body(e_ref, hh_ref, ht_ref, wpre_ref, wf_ref, wb_ref,
             be_ref, ge_ref, bee_ref, bf_ref, bb_ref,
             en_out, mf_out, mb_out):
        ev = e_ref[...]
        hhv = hh_ref[...]
        htv = ht_ref[...]
        x = jnp.concatenate([hhv, ev, htv], axis=-1)
        pre = jnp.dot(x, wpre_ref[...], preferred_element_type=jnp.float32)
        u = _leaky(pre + be_ref[...]) + ev
        en = _ln(u, ge_ref[...], bee_ref[...])
        en_out[...] = en
        xf = jnp.concatenate([hhv, en], axis=-1)
        mf_out[...] = (
            jnp.dot(xf, wf_ref[...], preferred_element_type=jnp.float32)
            + bf_ref[...]
        )
        xb = jnp.concatenate([htv, en], axis=-1)
        mb_out[...] = (
            jnp.dot(xb, wb_ref[...], preferred_element_type=jnp.float32)
            + bb_ref[...]
        )

    outs = [jax.ShapeDtypeStruct((m, d), jnp.float32)] * 3
    return pl.pallas_call(
        body,
        grid=(m // bm,),
        in_specs=[
            pl.BlockSpec((bm, d), lambda i: (i, 0)),
            pl.BlockSpec((bm, d), lambda i: (i, 0)),
            pl.BlockSpec((bm, d), lambda i: (i, 0)),
            pl.BlockSpec((3 * d, d), lambda i: (0, 0)),
            pl.BlockSpec((2 * d, d), lambda i: (0, 0)),
            pl.BlockSpec((2 * d, d), lambda i: (0, 0)),
            pl.BlockSpec((1, d), lambda i: (0, 0)),
            pl.BlockSpec((1, d), lambda i: (0, 0)),
            pl.BlockSpec((1, d), lambda i: (0, 0)),
            pl.BlockSpec((1, d), lambda i: (0, 0)),
            pl.BlockSpec((1, d), lambda i: (0, 0)),
        ],
        out_specs=[
            pl.BlockSpec((bm, d), lambda i: (i, 0)),
            pl.BlockSpec((bm, d), lambda i: (i, 0)),
            pl.BlockSpec((bm, d), lambda i: (i, 0)),
        ],
        out_shape=outs,
    )(e, hh, ht, wpre, wf, wb, be, ge, bee, bf, bb)


def _node_call(h, accs, ct, ch, gn, bn_):
    n, d = h.shape
    bn = 1000
    na = len(accs)

    def body(*refs):
        h_ref = refs[0]
        acc_refs = refs[1 : 1 + na]
        ct_ref, ch_ref, gn_ref, bn_ref, h_out = refs[1 + na :]
        agg = acc_refs[0][...]
        for r in acc_refs[1:]:
            agg = agg + r[...]
        cnt = ct_ref[...][:, :1] + ch_ref[...][:, :1]
        agg = agg / cnt
        u = _leaky(agg) + h_ref[...]
        h_out[...] = _ln(u, gn_ref[...], bn_ref[...])

    return pl.pallas_call(
        body,
        grid=(n // bn,),
        in_specs=[pl.BlockSpec((bn, d), lambda i: (i, 0))] * (1 + na)
        + [
            pl.BlockSpec((bn, d), lambda i: (i, 0)),
            pl.BlockSpec((bn, d), lambda i: (i, 0)),
            pl.BlockSpec((1, d), lambda i: (0, 0)),
            pl.BlockSpec((1, d), lambda i: (0, 0)),
        ],
        out_specs=pl.BlockSpec((bn, d), lambda i: (i, 0)),
        out_shape=jax.ShapeDtypeStruct((n, d), jnp.float32),
    )(h, *accs, ct, ch, gn, bn_)


def kernel(H, E, r_embed, ht, queries, layers):
    n, d = H.shape
    m = E.shape[0]
    r_rows = m // 128

    heads = ht[:, 0].astype(jnp.int32)
    tails = ht[:, 1].astype(jnp.int32)
    h2d = heads.reshape(r_rows, 128)
    t2d = tails.reshape(r_rows, 128)
    pad = PAD_ROWS - r_rows
    hpn = jnp.pad(h2d, ((0, pad), (0, 0)), constant_values=n)
    tpn = jnp.pad(t2d, ((0, pad), (0, 0)), constant_values=n)

    nch = 1
    crows = r_rows // nch
    prow = PAD_ROWS // nch
    cidx = []
    for c in range(nch):
        hc = h2d[c * crows : (c + 1) * crows]
        tc = t2d[c * crows : (c + 1) * crows]
        cpad = prow - crows
        cidx.append(
            dict(
                hp0=jnp.pad(hc, ((0, cpad), (0, 0))),
                tp0=jnp.pad(tc, ((0, cpad), (0, 0))),
                hpn=jnp.pad(hc, ((0, cpad), (0, 0)), constant_values=n),
                tpn=jnp.pad(tc, ((0, cpad), (0, 0)), constant_values=n),
            )
        )

    zrows = jnp.zeros((n // NS, d), jnp.float32)
    zc = jnp.zeros((n // NS, 128), jnp.float32)

    packed = []
    for lp in layers:
        packed.append(
            dict(
                wpre=lp["We"].T,
                wf=lp["Wf"].T,
                wb=lp["Wb"].T,
                be=lp["be"].reshape(1, d),
                ge=lp["ge"].reshape(1, d),
                bee=lp["bee"].reshape(1, d),
                bf=lp["bf"].reshape(1, d),
                bb=lp["bb"].reshape(1, d),
                gn=lp["gn"].reshape(1, d),
                bn=lp["bn"].reshape(1, d),
            )
        )

    ct, ch = _counts_call(tpn, hpn, zc)

    cm = crows * 128
    echunks = [E[c * cm : (c + 1) * cm] for c in range(nch)]
    for lp in packed:
        accs = []
        enchunks = []
        for c in range(nch):
            ci = cidx[c]
            hh, htg = _gather_call(H, ci["hp0"], ci["tp0"], crows, prow)
            en, mf, mb = _edge_msg_call(
                echunks[c], hh, htg,
                lp["wpre"], lp["wf"], lp["wb"],
                lp["be"], lp["ge"], lp["bee"], lp["bf"], lp["bb"],
            )
            af, ab = _scatter_m_call(
                mf, mb, ci["tpn"], ci["hpn"], zrows, crows, prow
            )
            accs += [af, ab]
            enchunks.append(en)
        H = _node_call(H, accs, ct, ch, lp["gn"], lp["bn"])
        echunks = enchunks
    return H


# speedup vs baseline: 1.0096x; 1.0096x over previous
import jax
import jax.numpy as jnp
from jax import lax
from jax.experimental import pallas as pl
from jax.experimental.pallas import tpu as pltpu
from jax.experimental.pallas import tpu_sc as plsc

D = 128
NS = 16
NC = 2
GR = 2
EG = GR * 128
PAD_ROWS = 2560


def _sc_mesh():
    return plsc.VectorSubcoreMesh(core_axis_name="c", subcore_axis_name="s")


def _sc_params():
    return pltpu.CompilerParams(use_tc_tiling_on_sc=False)


def _counts_call(tpn, hpn, zc):
    n = zc.shape[0] * NS
    nper = n // NS
    nbatch = PAD_ROWS // 8 // NS

    def body(t_ref, h_ref, zc_ref, ct_ref, ch_ref, acc, idxb, ones, sem):
        cid = lax.axis_index("c")
        sid = lax.axis_index("s")

        def fill(i, _):
            for jj in range(8):
                ones[i, pl.ds(jj * 16, 16)] = jnp.full((16,), 1.0, jnp.float32)
            return 0

        lax.fori_loop(0, 128, fill, 0)
        pltpu.sync_copy(zc_ref, acc.at[pl.ds(sid * nper, nper)])
        plsc.subcore_barrier()

        def stream(idx_ref):
            def batch(b, _):
                r0 = (b * NS + sid) * 8
                pltpu.sync_copy(idx_ref.at[pl.ds(r0, 8)], idxb)
                cps = [
                    pltpu.async_copy(ones, acc.at[idxb.at[j]], sem, add=True)
                    for j in range(8)
                ]
                for cp in cps:
                    cp.wait()
                return 0

            lax.fori_loop(0, nbatch, batch, 0)

        @pl.when(cid == 0)
        def _():
            stream(t_ref)

        @pl.when(cid == 1)
        def _():
            stream(h_ref)

        plsc.subcore_barrier()

        @pl.when(cid == 0)
        def _():
            pltpu.sync_copy(
                acc.at[pl.ds(sid * nper, nper)], ct_ref.at[pl.ds(sid * nper, nper)]
            )

        @pl.when(cid == 1)
        def _():
            pltpu.sync_copy(
                acc.at[pl.ds(sid * nper, nper)], ch_ref.at[pl.ds(sid * nper, nper)]
            )

    out = jax.ShapeDtypeStruct((n, 128), jnp.float32)
    return pl.kernel(
        body,
        out_type=(out, out),
        mesh=_sc_mesh(),
        compiler_params=_sc_params(),
        scratch_types=[
            pltpu.VMEM_SHARED((n + 16, 128), jnp.float32),
            pltpu.VMEM((8, 128), jnp.int32),
            pltpu.VMEM((128, 128), jnp.float32),
            pltpu.SemaphoreType.DMA,
        ],
    )(tpn, hpn, zc)


def _gather_call(h, hp0, tp0, r_rows, prow):
    n, d = h.shape
    m = r_rows * 128
    ng = (r_rows + GR - 1) // GR
    nloop = prow // GR // NS

    def body(h_ref, hidx_ref, tidx_ref, hh_ref, ht_ref,
             idx0, idx1, rows0, rows1, sem0, sem1):
        cid = lax.axis_index("c")
        sid = lax.axis_index("s")

        def stream(idx_ref, out_ref):
            bufs = ((idx0, rows0, sem0), (idx1, rows1, sem1))

            def prep(i, pr):
                idxb, rows, sem = bufs[pr]
                g = i * NS + sid
                pltpu.sync_copy(idx_ref.at[pl.ds(g * GR, GR)], idxb)
                for j in range(GR):
                    pltpu.async_copy(
                        h_ref.at[idxb.at[j]],
                        rows.at[pl.ds(j * 128, 128)],
                        sem,
                    )

            def finish(i, pr):
                idxb, rows, sem = bufs[pr]
                g = i * NS + sid
                for j in range(GR):
                    pltpu.make_async_copy(
                        h_ref.at[idxb.at[j]],
                        rows.at[pl.ds(j * 128, 128)],
                        sem,
                    ).wait()

                @pl.when(g < ng)
                def _():
                    pltpu.sync_copy(rows, out_ref.at[pl.ds(g * EG, EG)])

            prep(0, 0)

            def it(k, _):
                i0 = k * 2
                prep(i0 + 1, 1)
                finish(i0, 0)

                @pl.when(i0 + 2 < nloop)
                def _():
                    prep(i0 + 2, 0)

                finish(i0 + 1, 1)
                return 0

            lax.fori_loop(0, nloop // 2, it, 0)

        @pl.when(cid == 0)
        def _():
            stream(hidx_ref, hh_ref)

        @pl.when(cid == 1)
        def _():
            stream(tidx_ref, ht_ref)

    out = jax.ShapeDtypeStruct((m, d), jnp.float32)
    return pl.kernel(
        body,
        out_type=(out, out),
        mesh=_sc_mesh(),
        compiler_params=_sc_params(),
        scratch_types=[
            pltpu.VMEM((GR, 128), jnp.int32),
            pltpu.VMEM((GR, 128), jnp.int32),
            pltpu.VMEM((EG, d), jnp.float32),
            pltpu.VMEM((EG, d), jnp.float32),
            pltpu.SemaphoreType.DMA,
            pltpu.SemaphoreType.DMA,
        ],
    )(h, hp0, tp0)


def _scatter_m_call(mf, mb, tpn, hpn, zrows, r_rows, prow):
    m, d = mf.shape
    n = zrows.shape[0] * NS
    nper = n // NS
    nloop = prow // NS
    t32 = tpn.reshape(prow * 4, 32)
    h32 = hpn.reshape(prow * 4, 32)

    def body(mf_ref, mb_ref, t_ref, h_ref, z_ref, af_ref, ab_ref,
             acc, idx0, idx1, rows0, rows1, lsem0, lsem1, ssem0, ssem1):
        cid = lax.axis_index("c")
        sid = lax.axis_index("s")
        pltpu.sync_copy(z_ref, acc.at[pl.ds(sid * nper, nper)])
        plsc.subcore_barrier()

        def stream(src_ref, idx_ref):
            g0_ = sid * nloop
            bufs = (
                (idx0, rows0, lsem0, ssem0),
                (idx1, rows1, lsem1, ssem1),
            )

            def prep(i, pr):
                ib, rows, lsem, _ = bufs[pr]
                g = g0_ + i
                r = jnp.where(g < r_rows, g, 0)
                pltpu.sync_copy(idx_ref.at[pl.ds(4 * g, 4)], ib)
                pltpu.async_copy(src_ref.at[pl.ds(r * 128, 128)], rows, lsem)

            def launch(i, pr):
                ib, rows, lsem, ssem = bufs[pr]
                pltpu.make_async_copy(
                    src_ref.at[pl.ds(0, 128)], rows, lsem
                ).wait()
                for j in range(4):
                    pltpu.async_copy(
                        rows.at[pl.ds(j * 32, 32)],
                        acc.at[ib.at[j]],
                        ssem,
                        add=True,
                    )

            def drain(pr):
                ib, rows, _, ssem = bufs[pr]
                for j in range(4):
                    pltpu.make_async_copy(
                        rows.at[pl.ds(j * 32, 32)], acc.at[ib.at[j]], ssem
                    ).wait()

            prep(0, 0)

            def it(k, _):
                i0 = k * 2
                launch(i0, 0)

                @pl.when(k > 0)
                def _():
                    drain(1)

                prep(i0 + 1, 1)
                drain(0)

                @pl.when(i0 + 2 < nloop)
                def _():
                    prep(i0 + 2, 0)

                launch(i0 + 1, 1)
                return 0

            lax.fori_loop(0, nloop // 2, it, 0)
            drain(1)

        @pl.when(cid == 0)
        def _():
            stream(mf_ref, t_ref)

        @pl.when(cid == 1)
        def _():
            stream(mb_ref, h_ref)

        plsc.subcore_barrier()

        @pl.when(cid == 0)
        def _():
            pltpu.sync_copy(
                acc.at[pl.ds(sid * nper, nper)], af_ref.at[pl.ds(sid * nper, nper)]
            )

        @pl.when(cid == 1)
        def _():
            pltpu.sync_copy(
                acc.at[pl.ds(sid * nper, nper)], ab_ref.at[pl.ds(sid * nper, nper)]
            )

    out = jax.ShapeDtypeStruct((n, d), jnp.float32)
    return pl.kernel(
        body,
        out_type=(out, out),
        mesh=_sc_mesh(),
        compiler_params=_sc_params(),
        scratch_types=[
            pltpu.VMEM_SHARED((n + 16, d), jnp.float32),
            pltpu.VMEM((4, 32), jnp.int32),
            pltpu.VMEM((4, 32), jnp.int32),
            pltpu.VMEM((128, d), jnp.float32),
            pltpu.VMEM((128, d), jnp.float32),
            pltpu.SemaphoreType.DMA,
            pltpu.SemaphoreType.DMA,
            pltpu.SemaphoreType.DMA,
            pltpu.SemaphoreType.DMA,
        ],
    )(mf, mb, t32, h32, zrows)


def _leaky(x):
    return jnp.where(x >= 0, x, 0.01 * x)


def _ln(x, g, b):
    mu = jnp.mean(x, axis=-1, keepdims=True)
    var = jnp.mean((x - mu) ** 2, axis=-1, keepdims=True)
    return (x - mu) * lax.rsqrt(var + 1e-5) * g + b


def _edge_msg_call(e, hh, ht, wpre, wf, wb, be, ge, bee, bf, bb):
    m, d = e.shape
    bm = 3200

    def body(e_ref, hh_ref, ht_ref, wpre_ref, wf_ref, wb_ref,
             be_ref, ge_ref, bee_ref, bf_ref, bb_ref,
             en_out, mf_out, mb_out):
        ev = e_ref[...]
        hhv = hh_ref[...]
        htv = ht_ref[...]
        x = jnp.concatenate([hhv, ev, htv], axis=-1)
        pre = jnp.dot(x, wpre_ref[...], preferred_element_type=jnp.float32)
        u = _leaky(pre + be_ref[...]) + ev
        en = _ln(u, ge_ref[...], bee_ref[...])
        en_out[...] = en
        xf = jnp.concatenate([hhv, en], axis=-1)
        mf_out[...] = (
            jnp.dot(xf, wf_ref[...], preferred_element_type=jnp.float32)
            + bf_ref[...]
        )
        xb = jnp.concatenate([htv, en], axis=-1)
        mb_out[...] = (
            jnp.dot(xb, wb_ref[...], preferred_element_type=jnp.float32)
            + bb_ref[...]
        )

    outs = [jax.ShapeDtypeStruct((m, d), jnp.float32)] * 3
    return pl.pallas_call(
        body,
        grid=(m // bm,),
        in_specs=[
            pl.BlockSpec((bm, d), lambda i: (i, 0)),
            pl.BlockSpec((bm, d), lambda i: (i, 0)),
            pl.BlockSpec((bm, d), lambda i: (i, 0)),
            pl.BlockSpec((3 * d, d), lambda i: (0, 0)),
            pl.BlockSpec((2 * d, d), lambda i: (0, 0)),
            pl.BlockSpec((2 * d, d), lambda i: (0, 0)),
            pl.BlockSpec((1, d), lambda i: (0, 0)),
            pl.BlockSpec((1, d), lambda i: (0, 0)),
            pl.BlockSpec((1, d), lambda i: (0, 0)),
            pl.BlockSpec((1, d), lambda i: (0, 0)),
            pl.BlockSpec((1, d), lambda i: (0, 0)),
        ],
        out_specs=[
            pl.BlockSpec((bm, d), lambda i: (i, 0)),
            pl.BlockSpec((bm, d), lambda i: (i, 0)),
            pl.BlockSpec((bm, d), lambda i: (i, 0)),
        ],
        out_shape=outs,
    )(e, hh, ht, wpre, wf, wb, be, ge, bee, bf, bb)


def _node_call(h, accs, ct, ch, gn, bn_):
    n, d = h.shape
    bn = 1000
    na = len(accs)

    def body(*refs):
        h_ref = refs[0]
        acc_refs = refs[1 : 1 + na]
        ct_ref, ch_ref, gn_ref, bn_ref, h_out = refs[1 + na :]
        agg = acc_refs[0][...]
        for r in acc_refs[1:]:
            agg = agg + r[...]
        cnt = ct_ref[...][:, :1] + ch_ref[...][:, :1]
        agg = agg / cnt
        u = _leaky(agg) + h_ref[...]
        h_out[...] = _ln(u, gn_ref[...], bn_ref[...])

    return pl.pallas_call(
        body,
        grid=(n // bn,),
        in_specs=[pl.BlockSpec((bn, d), lambda i: (i, 0))] * (1 + na)
        + [
            pl.BlockSpec((bn, d), lambda i: (i, 0)),
            pl.BlockSpec((bn, d), lambda i: (i, 0)),
            pl.BlockSpec((1, d), lambda i: (0, 0)),
            pl.BlockSpec((1, d), lambda i: (0, 0)),
        ],
        out_specs=pl.BlockSpec((bn, d), lambda i: (i, 0)),
        out_shape=jax.ShapeDtypeStruct((n, d), jnp.float32),
    )(h, *accs, ct, ch, gn, bn_)


def kernel(H, E, r_embed, ht, queries, layers):
    n, d = H.shape
    m = E.shape[0]
    r_rows = m // 128

    heads = ht[:, 0].astype(jnp.int32)
    tails = ht[:, 1].astype(jnp.int32)
    h2d = heads.reshape(r_rows, 128)
    t2d = tails.reshape(r_rows, 128)
    pad = PAD_ROWS - r_rows
    hpn = jnp.pad(h2d, ((0, pad), (0, 0)), constant_values=n)
    tpn = jnp.pad(t2d, ((0, pad), (0, 0)), constant_values=n)

    nch = 1
    crows = r_rows // nch
    prow = PAD_ROWS // nch
    cidx = []
    for c in range(nch):
        hc = h2d[c * crows : (c + 1) * crows]
        tc = t2d[c * crows : (c + 1) * crows]
        cpad = prow - crows
        cidx.append(
            dict(
                hp0=jnp.pad(hc, ((0, cpad), (0, 0))),
                tp0=jnp.pad(tc, ((0, cpad), (0, 0))),
                hpn=jnp.pad(hc, ((0, cpad), (0, 0)), constant_values=n),
                tpn=jnp.pad(tc, ((0, cpad), (0, 0)), constant_values=n),
            )
        )

    zrows = jnp.zeros((n // NS, d), jnp.float32)
    zc = jnp.zeros((n // NS, 128), jnp.float32)

    packed = []
    for lp in layers:
        packed.append(
            dict(
                wpre=lp["We"].T,
                wf=lp["Wf"].T,
                wb=lp["Wb"].T,
                be=lp["be"].reshape(1, d),
                ge=lp["ge"].reshape(1, d),
                bee=lp["bee"].reshape(1, d),
                bf=lp["bf"].reshape(1, d),
                bb=lp["bb"].reshape(1, d),
                gn=lp["gn"].reshape(1, d),
                bn=lp["bn"].reshape(1, d),
            )
        )

    ct, ch = _counts_call(tpn, hpn, zc)

    cm = crows * 128
    echunks = [E[c * cm : (c + 1) * cm] for c in range(nch)]
    for lp in packed:
        accs = []
        enchunks = []
        for c in range(nch):
            ci = cidx[c]
            hh, htg = _gather_call(H, ci["hp0"], ci["tp0"], crows, prow)
            en, mf, mb = _edge_msg_call(
                echunks[c], hh, htg,
                lp["wpre"], lp["wf"], lp["wb"],
                lp["be"], lp["ge"], lp["bee"], lp["bf"], lp["bb"],
            )
            af, ab = _scatter_m_call(
                mf, mb, ci["tpn"], ci["hpn"], zrows, crows, prow
            )
            accs += [af, ab]
            enchunks.append(en)
        H = _node_call(H, accs, ct, ch, lp["gn"], lp["bn"])
        echunks = enchunks
    return H
